# baseline (speedup 1.0000x reference)
import functools
import jax
import jax.numpy as jnp
from jax import lax
from jax.experimental import pallas as pl
from jax.experimental.pallas import tpu as pltpu
from jax.experimental.pallas import tpu_sc as plsc

N = 32768
D = 512
H = 1024
NB = 16
BLK = 2048

NC, NS = 2, 16
NW = NC * NS
CHUNK = N // NW


def _sc_counts_body(ids_hbm, ones_hbm, zeros_hbm, out_hbm, idx_v, ones_v,
                    hist_v, tmp_v, shared):
    c = lax.axis_index("c")
    s = lax.axis_index("s")
    wid = c * NS + s
    pltpu.sync_copy(ids_hbm.at[pl.ds(wid * CHUNK, CHUNK)], idx_v)
    pltpu.sync_copy(ones_hbm.at[pl.ds(0, CHUNK)], ones_v)
    pltpu.sync_copy(zeros_hbm, shared.at[s])
    pltpu.sync_copy(ones_v, shared.at[s].at[idx_v], add=True)
    plsc.subcore_barrier()

    @pl.when(s == 0)
    def _reduce():
        pltpu.sync_copy(zeros_hbm, hist_v)
        for r in range(NS - 1, -1, -1):
            pltpu.sync_copy(shared.at[r], tmp_v)
            for j in range(8):
                sl = pl.ds(j * 16, 16)
                hist_v[sl] = hist_v[sl] + tmp_v[sl]
        pltpu.sync_copy(hist_v, out_hbm.at[c])


def _sc_counts(ids32):
    mesh = plsc.VectorSubcoreMesh(core_axis_name="c", subcore_axis_name="s")
    k = functools.partial(
        pl.kernel,
        mesh=mesh,
        out_type=jax.ShapeDtypeStruct((NC, 128), jnp.float32),
        scratch_types=[
            pltpu.VMEM((CHUNK,), jnp.int32),
            pltpu.VMEM((CHUNK,), jnp.float32),
            pltpu.VMEM((128,), jnp.float32),
            pltpu.VMEM((128,), jnp.float32),
            pltpu.VMEM_SHARED((NS, 128), jnp.float32),
        ],
    )(_sc_counts_body)
    return k(ids32, jnp.ones((CHUNK,), jnp.float32), jnp.zeros((128,), jnp.float32))


def _sums_body(ids_ref, x_ref, w1_ref, b1_ref, out_ref, acc_ref):
    i = pl.program_id(0)
    nsteps = pl.num_programs(0)

    @pl.when(i == 0)
    def _init():
        acc_ref[...] = jnp.zeros_like(acc_ref)

    h = jnp.dot(x_ref[...], w1_ref[...], preferred_element_type=jnp.float32)
    h = jnp.maximum(h + b1_ref[...], 0.0)

    ids_blk = ids_ref[0, :]
    onehot = (ids_blk[None, :] ==
              jax.lax.broadcasted_iota(jnp.int32, (NB, BLK), 0)).astype(jnp.float32)
    acc_ref[...] += jnp.dot(onehot, h, preferred_element_type=jnp.float32)

    @pl.when(i == nsteps - 1)
    def _flush():
        out_ref[...] = acc_ref[...]


def _finish_body(acc_ref, pc_ref, w2_ref, b2_ref, w3_ref, b3_ref, out_ref):
    pcs = pc_ref[0:1, :] + pc_ref[1:2, :]
    sel = (jax.lax.broadcasted_iota(jnp.int32, (NB, 128), 0) ==
           jax.lax.broadcasted_iota(jnp.int32, (NB, 128), 1)).astype(jnp.float32)
    cnt = jax.lax.dot_general(sel, pcs, (((1,), (1,)), ((), ())),
                              preferred_element_type=jnp.float32)
    cnt = jnp.maximum(cnt, 1.0)
    s = jnp.dot(acc_ref[...], w2_ref[...], preferred_element_type=jnp.float32)
    agg = s / cnt + b2_ref[...]
    out_ref[...] = jnp.dot(agg, w3_ref[...], preferred_element_type=jnp.float32) + b3_ref[...]


def kernel(x, ids, W1, b1, W2, b2, W3, b3):
    inner_ids = ids[-1].astype(jnp.int32)
    pcounts = _sc_counts(inner_ids)

    grid = (N // BLK,)
    acc = pl.pallas_call(
        _sums_body,
        grid=grid,
        in_specs=[
            pl.BlockSpec((1, BLK), lambda i: (0, i)),
            pl.BlockSpec((BLK, D), lambda i: (i, 0)),
            pl.BlockSpec((D, H), lambda i: (0, 0)),
            pl.BlockSpec((1, H), lambda i: (0, 0)),
        ],
        out_specs=pl.BlockSpec((NB, H), lambda i: (0, 0)),
        out_shape=jax.ShapeDtypeStruct((NB, H), jnp.float32),
        scratch_shapes=[pltpu.VMEM((NB, H), jnp.float32)],
        compiler_params=pltpu.CompilerParams(
            dimension_semantics=("arbitrary",),
        ),
    )(inner_ids.reshape(1, N), x, W1, b1.reshape(1, H))

    out = pl.pallas_call(
        _finish_body,
        out_shape=jax.ShapeDtypeStruct((NB, 128), jnp.float32),
    )(acc, pcounts, W2, b2.reshape(1, D), W3, b3.reshape(1, 128))
    return out


# speedup vs baseline: 1.3509x; 1.3509x over previous
import jax
import jax.numpy as jnp
from jax.experimental import pallas as pl
from jax.experimental.pallas import tpu as pltpu

N = 32768
D = 512
H = 1024
NB = 16
BLK = 2048


def _fused_body(ids_ref, x_ref, w1_ref, b1_ref, w2_ref, b2_ref, w3_ref, b3_ref,
                out_ref, acc_ref, cnt_ref):
    i = pl.program_id(0)
    nsteps = pl.num_programs(0)

    @pl.when(i == 0)
    def _init():
        acc_ref[...] = jnp.zeros_like(acc_ref)
        cnt_ref[...] = jnp.zeros_like(cnt_ref)

    h = jnp.dot(x_ref[...], w1_ref[...], preferred_element_type=jnp.float32)
    h = jnp.maximum(h + b1_ref[...], 0.0)

    ids_blk = ids_ref[0, :]
    onehot = (ids_blk[None, :] ==
              jax.lax.broadcasted_iota(jnp.int32, (NB, BLK), 0)).astype(jnp.float32)
    acc_ref[...] += jnp.dot(onehot, h, preferred_element_type=jnp.float32)
    cnt_ref[...] += jnp.sum(onehot, axis=1, keepdims=True)

    @pl.when(i == nsteps - 1)
    def _finish():
        cnt = jnp.maximum(cnt_ref[:, :1], 1.0)
        s = jnp.dot(acc_ref[...], w2_ref[...], preferred_element_type=jnp.float32)
        agg = s / cnt + b2_ref[...]
        out_ref[...] = jnp.dot(agg, w3_ref[...], preferred_element_type=jnp.float32) + b3_ref[...]


def kernel(x, ids, W1, b1, W2, b2, W3, b3):
    inner_ids = ids[-1].astype(jnp.int32).reshape(1, N)
    grid = (N // BLK,)
    out = pl.pallas_call(
        _fused_body,
        grid=grid,
        in_specs=[
            pl.BlockSpec((1, BLK), lambda i: (0, i)),
            pl.BlockSpec((BLK, D), lambda i: (i, 0)),
            pl.BlockSpec((D, H), lambda i: (0, 0)),
            pl.BlockSpec((1, H), lambda i: (0, 0)),
            pl.BlockSpec((H, D), lambda i: (0, 0)),
            pl.BlockSpec((1, D), lambda i: (0, 0)),
            pl.BlockSpec((D, 128), lambda i: (0, 0)),
            pl.BlockSpec((1, 128), lambda i: (0, 0)),
        ],
        out_specs=pl.BlockSpec((NB, 128), lambda i: (0, 0)),
        out_shape=jax.ShapeDtypeStruct((NB, 128), jnp.float32),
        scratch_shapes=[
            pltpu.VMEM((NB, H), jnp.float32),
            pltpu.VMEM((NB, 128), jnp.float32),
        ],
        compiler_params=pltpu.CompilerParams(
            dimension_semantics=("arbitrary",),
        ),
    )(inner_ids, x, W1, b1.reshape(1, H), W2, b2.reshape(1, D), W3, b3.reshape(1, 128))
    return out
